# baseline (speedup 1.0000x reference)
import functools

import jax
import jax.numpy as jnp
from jax import lax
from jax.experimental import pallas as pl
from jax.experimental.pallas import tpu as pltpu
from jax.experimental.pallas import tpu_sc as plsc

N = 10000
E = 160000
D = 256
HALF = 128
AUGW = 144
NC = 2
NS = 16
LANES = 16
B = 128
NB_PT = 80
E_PAD = NB_PT * NS * B
ROWS_PAD = E_PAD // B
NPT = N // NS
BLK = 1000
NBLK = N // BLK


def _make_sc_aggregate(width):
  mesh = plsc.VectorSubcoreMesh(
      core_axis_name="c", subcore_axis_name="s", num_cores=NC, num_subcores=NS)

  @functools.partial(
      pl.kernel,
      out_type=jax.ShapeDtypeStruct((NC * N, width), jnp.float32),
      mesh=mesh,
      scratch_types=[
          pltpu.VMEM_SHARED((N, width), jnp.float32),
          pltpu.VMEM((B, width), jnp.float32),
      ] + [pltpu.VMEM((B, HALF), jnp.bfloat16) for _ in range(2)]
        + [pltpu.VMEM((4, B), jnp.int32) for _ in range(2)]
        + [pltpu.SemaphoreType.DMA for _ in range(4)],
      compiler_params=pltpu.CompilerParams(
          use_tc_tiling_on_sc=False, needs_layout_passes=False),
  )
  def agg(table_hbm, packed_hbm, zeros_hbm, out_hbm, acc, frows, *scr):
    rows = scr[0:2]
    idxb = scr[2:4]
    gsem = scr[4:6]
    isem = scr[6:8]
    c = lax.axis_index("c")
    s = lax.axis_index("s")
    base_r = s * NB_PT

    pltpu.sync_copy(zeros_hbm, acc.at[pl.ds(s * NPT, NPT)])
    plsc.subcore_barrier()

    def idx_start(j, b):
      pltpu.async_copy(packed_hbm.at[base_r + b], idxb[j], isem[j])

    def idx_wait(j, b):
      pltpu.make_async_copy(packed_hbm.at[base_r + b], idxb[j],
                            isem[j]).wait()

    def gather_start(j):
      pltpu.async_copy(table_hbm.at[idxb[j].at[c]], rows[j], gsem[j])

    def gather_wait(j):
      pltpu.make_async_copy(table_hbm.at[idxb[j].at[c]], rows[j],
                            gsem[j]).wait()

    onehot = jnp.where(lax.iota(jnp.int32, LANES) == 0, 1.0, 0.0)

    def scale(j):
      rj = rows[j]
      wj = idxb[j]

      def mul_chunk(kb, carry):
        kbase = kb * LANES
        wk_vec = wj[3, pl.ds(kbase, LANES)]
        for l in range(LANES):
          wk = lax.bitcast_convert_type(wk_vec[l], jnp.float32)
          for g in range(HALF // (2 * LANES)):
            hv = rj[kbase + l, pl.ds(g * 2 * LANES, 2 * LANES)]
            lo, hi = plsc.unpack(hv, format=plsc.PackFormat.INTERLEAVED)
            frows[kbase + l, pl.ds(g * 2 * LANES, LANES)] = lo * wk
            frows[kbase + l, pl.ds(g * 2 * LANES + LANES, LANES)] = hi * wk
          if width > HALF:
            frows[kbase + l, pl.ds(HALF, LANES)] = onehot * wk
        return carry

      lax.fori_loop(0, B // LANES, mul_chunk, 0)

    idx_start(0, 0)
    idx_wait(0, 0)
    gather_start(0)
    idx_start(1, 1)

    def body(i, carry):
      for j in range(2):
        b = 2 * i + j
        p = 1 - j
        gather_wait(j)

        @pl.when(b + 1 < NB_PT)
        def _():
          idx_wait(p, b + 1)
          gather_start(p)

        scale(j)

        pltpu.sync_copy(frows, acc.at[idxb[j].at[2]], add=True)

        @pl.when(b + 2 < NB_PT)
        def _():
          idx_start(j, b + 2)
      return carry

    lax.fori_loop(0, NB_PT // 2, body, 0)
    plsc.subcore_barrier()
    pltpu.sync_copy(acc.at[pl.ds(s * NPT, NPT)],
                    out_hbm.at[pl.ds(c * N + s * NPT, NPT)])

  return agg


_sc_agg_aug = _make_sc_aggregate(AUGW)
_sc_agg_half = _make_sc_aggregate(HALF)


def _tc_layer1(a1, w1, b1):
  def body(aa_ref, ab_ref, w1a_ref, w1b_ref, b1_ref, out_ref):
    aa = aa_ref[...]
    ab = ab_ref[...]
    deg = aa[:, HALF:HALF + 1]
    sc = jnp.where(deg > 0.0, 1.0 / deg, 0.0)
    xa = aa[:, :HALF] * sc
    xb = ab[:, :HALF] * sc
    h = (jnp.dot(xa, w1a_ref[...], preferred_element_type=jnp.float32)
         + jnp.dot(xb, w1b_ref[...], preferred_element_type=jnp.float32)
         + b1_ref[...])
    out_ref[...] = jnp.maximum(h, 0.0)

  return pl.pallas_call(
      body,
      grid=(2, NBLK),
      in_specs=[
          pl.BlockSpec((BLK, AUGW), lambda j, i: (i, 0)),
          pl.BlockSpec((BLK, AUGW), lambda j, i: (i + NBLK, 0)),
          pl.BlockSpec((HALF, HALF), lambda j, i: (0, j)),
          pl.BlockSpec((HALF, HALF), lambda j, i: (1, j)),
          pl.BlockSpec((1, HALF), lambda j, i: (0, j)),
      ],
      out_specs=pl.BlockSpec((BLK, HALF), lambda j, i: (j * NBLK + i, 0)),
      out_shape=jax.ShapeDtypeStruct((2 * N, HALF), jnp.float32),
      compiler_params=pltpu.CompilerParams(
          dimension_semantics=("parallel", "parallel")),
  )(a1, a1, w1, w1, b1.reshape(1, D))


def _tc_layer2(a2, a1, w2, b2, wd, bd, wc, bc):
  def body(a2a_ref, a2b_ref, dega_ref, w2a_ref, w2b_ref, b2_ref,
           wd_ref, bd_ref, wc_ref, bc_ref, out_ref, acc_ref):
    i = pl.program_id(0)

    @pl.when(i == 0)
    def _():
      acc_ref[...] = jnp.zeros_like(acc_ref)

    deg = dega_ref[...][:, HALF:HALF + 1]
    sc = jnp.where(deg > 0.0, 1.0 / deg, 0.0)
    xa = a2a_ref[...] * sc
    xb = a2b_ref[...] * sc
    h2 = (jnp.dot(xa, w2a_ref[...], preferred_element_type=jnp.float32)
          + jnp.dot(xb, w2b_ref[...], preferred_element_type=jnp.float32)
          + b2_ref[...])
    h2 = jnp.maximum(h2, 0.0)
    acc_ref[...] += jnp.sum(h2, axis=0, keepdims=True)

    @pl.when(i == NBLK - 1)
    def _():
      hg = acc_ref[...] * (1.0 / N)
      o1 = jnp.maximum(
          jnp.dot(hg, wd_ref[...], preferred_element_type=jnp.float32)
          + bd_ref[...], 0.0)
      out_ref[...] = (
          jnp.dot(o1, wc_ref[...], preferred_element_type=jnp.float32)
          + bc_ref[...])

  return pl.pallas_call(
      body,
      grid=(NBLK,),
      in_specs=[
          pl.BlockSpec((BLK, HALF), lambda i: (i, 0)),
          pl.BlockSpec((BLK, HALF), lambda i: (i + NBLK, 0)),
          pl.BlockSpec((BLK, AUGW), lambda i: (i, 0)),
          pl.BlockSpec((HALF, D), lambda i: (0, 0)),
          pl.BlockSpec((HALF, D), lambda i: (1, 0)),
          pl.BlockSpec((1, D), lambda i: (0, 0)),
          pl.BlockSpec((D, HALF), lambda i: (0, 0)),
          pl.BlockSpec((1, HALF), lambda i: (0, 0)),
          pl.BlockSpec((HALF, 10), lambda i: (0, 0)),
          pl.BlockSpec((1, 10), lambda i: (0, 0)),
      ],
      out_specs=pl.BlockSpec((1, 10), lambda i: (0, 0)),
      out_shape=jax.ShapeDtypeStruct((1, 10), jnp.float32),
      scratch_shapes=[pltpu.VMEM((1, D), jnp.float32)],
      compiler_params=pltpu.CompilerParams(
          dimension_semantics=("arbitrary",)),
  )(a2, a2, a1, w2, w2, b2.reshape(1, D), wd, bd.reshape(1, HALF),
    wc, bc.reshape(1, 10))


def kernel(in_feat, edge_weights, W1, b1, W2, b2, Wd, bd, Wc, bc, edge_index):
  npad = E_PAD - E
  src = jnp.concatenate([edge_index[0], jnp.zeros((npad,), jnp.int32)])
  dst = jnp.concatenate([edge_index[1], jnp.zeros((npad,), jnp.int32)])
  w = jnp.concatenate([edge_weights, jnp.zeros((npad,), jnp.float32)])
  w_bits = lax.bitcast_convert_type(w, jnp.int32)
  packed = jnp.stack([src, src + N, dst, w_bits], axis=0)
  packed = packed.reshape(4, ROWS_PAD, B).transpose(1, 0, 2)

  def to_table(x):
    n = x.shape[0]
    xp = x.astype(jnp.bfloat16).reshape(n, HALF // 32, 2, LANES)
    return xp.transpose(0, 1, 3, 2).reshape(n, HALF)

  table1 = jnp.concatenate([
      to_table(in_feat[:, :HALF]),
      to_table(in_feat[:, HALF:]),
  ], axis=0)

  zeros_aug = jnp.zeros((NPT, AUGW), jnp.float32)
  zeros_half = jnp.zeros((NPT, HALF), jnp.float32)

  a1 = _sc_agg_aug(table1, packed, zeros_aug)
  h = _tc_layer1(a1, W1, b1)
  a2 = _sc_agg_half(to_table(h), packed, zeros_half)
  return _tc_layer2(a2, a1, W2, b2, Wd, bd, Wc, bc)


# speedup vs baseline: 1.2020x; 1.2020x over previous
import functools

import jax
import jax.numpy as jnp
from jax import lax
from jax.experimental import pallas as pl
from jax.experimental.pallas import tpu as pltpu
from jax.experimental.pallas import tpu_sc as plsc

N = 10000
E = 160000
D = 256
HALF = 128
AUGW = 144
NC = 2
NS = 16
LANES = 16
B = 128
NB_PT = 80
E_PAD = NB_PT * NS * B
ROWS_PAD = E_PAD // B
NPT = N // NS
BLK = 1000
NBLK = N // BLK


def _make_sc_aggregate(width):
  mesh = plsc.VectorSubcoreMesh(
      core_axis_name="c", subcore_axis_name="s", num_cores=NC, num_subcores=NS)

  @functools.partial(
      pl.kernel,
      out_type=jax.ShapeDtypeStruct((NC * N, width), jnp.float32),
      mesh=mesh,
      scratch_types=[
          pltpu.VMEM_SHARED((N, width), jnp.float32),
      ] + [pltpu.VMEM((B, width), jnp.float32) for _ in range(2)]
        + [pltpu.VMEM((4, B), jnp.int32) for _ in range(2)]
        + [pltpu.VMEM((B,), jnp.int32) for _ in range(2)]
        + [pltpu.SemaphoreType.DMA for _ in range(6)],
      compiler_params=pltpu.CompilerParams(use_tc_tiling_on_sc=False),
  )
  def agg(table_hbm, packed_hbm, zeros_hbm, out_hbm, acc, *scr):
    rows = scr[0:2]
    idxb = scr[2:4]
    dstb = scr[4:6]
    gsem = scr[6:8]
    isem = scr[8:10]
    ssem = scr[10:12]
    c = lax.axis_index("c")
    s = lax.axis_index("s")
    base_r = s * NB_PT

    pltpu.sync_copy(zeros_hbm, acc.at[pl.ds(s * NPT, NPT)])
    plsc.subcore_barrier()

    def idx_start(j, b):
      pltpu.async_copy(packed_hbm.at[base_r + b], idxb[j], isem[j])

    def idx_wait(j, b):
      pltpu.make_async_copy(packed_hbm.at[base_r + b], idxb[j],
                            isem[j]).wait()

    def gather_start(j):
      pltpu.async_copy(table_hbm.at[idxb[j].at[c]], rows[j], gsem[j])

    def gather_wait(j):
      pltpu.make_async_copy(table_hbm.at[idxb[j].at[c]], rows[j],
                            gsem[j]).wait()

    def dst_copy(j):
      for q in range(B // LANES):
        sl = pl.ds(q * LANES, LANES)
        dstb[j][sl] = idxb[j][2, sl]

    def scatter_start(j):
      pltpu.async_copy(rows[j], acc.at[dstb[j]], ssem[j], add=True)

    def scatter_wait(j):
      pltpu.make_async_copy(rows[j], acc.at[dstb[j]], ssem[j]).wait()

    def scale(j):
      rj = rows[j]
      wj = idxb[j]

      def mul_chunk(kb, carry):
        kbase = kb * LANES
        wk_vec = wj[3, pl.ds(kbase, LANES)]
        for l in range(LANES):
          wk = lax.bitcast_convert_type(wk_vec[l], jnp.float32)
          for q in range(width // LANES):
            sl = pl.ds(q * LANES, LANES)
            rj[kbase + l, sl] = rj[kbase + l, sl] * wk
        return carry

      lax.fori_loop(0, B // LANES, mul_chunk, 0)

    idx_start(0, 0)
    idx_wait(0, 0)
    gather_start(0)
    idx_start(1, 1)

    def body(i, carry):
      for j in range(2):
        b = 2 * i + j
        p = 1 - j
        gather_wait(j)
        dst_copy(j)

        @pl.when(b >= 1)
        def _():
          scatter_wait(p)

        @pl.when(b + 1 < NB_PT)
        def _():
          idx_wait(p, b + 1)
          gather_start(p)

        scale(j)
        scatter_start(j)

        @pl.when(b + 2 < NB_PT)
        def _():
          idx_start(j, b + 2)
      return carry

    lax.fori_loop(0, NB_PT // 2, body, 0)
    scatter_wait(1)
    plsc.subcore_barrier()
    pltpu.sync_copy(acc.at[pl.ds(s * NPT, NPT)],
                    out_hbm.at[pl.ds(c * N + s * NPT, NPT)])

  return agg


_sc_agg_aug = _make_sc_aggregate(AUGW)
_sc_agg_half = _make_sc_aggregate(HALF)


def _tc_layer1(a1, w1, b1):
  def body(aa_ref, ab_ref, w1a_ref, w1b_ref, b1_ref, out_ref):
    aa = aa_ref[...]
    ab = ab_ref[...]
    deg = aa[:, HALF:HALF + 1]
    sc = jnp.where(deg > 0.0, 1.0 / deg, 0.0)
    xa = aa[:, :HALF] * sc
    xb = ab[:, :HALF] * sc
    h = (jnp.dot(xa, w1a_ref[...], preferred_element_type=jnp.float32)
         + jnp.dot(xb, w1b_ref[...], preferred_element_type=jnp.float32)
         + b1_ref[...])
    out_ref[...] = jnp.maximum(h, 0.0)

  return pl.pallas_call(
      body,
      grid=(2, NBLK),
      in_specs=[
          pl.BlockSpec((BLK, AUGW), lambda j, i: (i, 0)),
          pl.BlockSpec((BLK, AUGW), lambda j, i: (i + NBLK, 0)),
          pl.BlockSpec((HALF, HALF), lambda j, i: (0, j)),
          pl.BlockSpec((HALF, HALF), lambda j, i: (1, j)),
          pl.BlockSpec((1, HALF), lambda j, i: (0, j)),
      ],
      out_specs=pl.BlockSpec((BLK, HALF), lambda j, i: (j * NBLK + i, 0)),
      out_shape=jax.ShapeDtypeStruct((2 * N, HALF), jnp.float32),
      compiler_params=pltpu.CompilerParams(
          dimension_semantics=("parallel", "parallel")),
  )(a1, a1, w1, w1, b1.reshape(1, D))


def _tc_layer2(a2, a1, w2, b2, wd, bd, wc, bc):
  def body(a2a_ref, a2b_ref, dega_ref, w2a_ref, w2b_ref, b2_ref,
           wd_ref, bd_ref, wc_ref, bc_ref, out_ref, acc_ref):
    i = pl.program_id(0)

    @pl.when(i == 0)
    def _():
      acc_ref[...] = jnp.zeros_like(acc_ref)

    deg = dega_ref[...][:, HALF:HALF + 1]
    sc = jnp.where(deg > 0.0, 1.0 / deg, 0.0)
    xa = a2a_ref[...] * sc
    xb = a2b_ref[...] * sc
    h2 = (jnp.dot(xa, w2a_ref[...], preferred_element_type=jnp.float32)
          + jnp.dot(xb, w2b_ref[...], preferred_element_type=jnp.float32)
          + b2_ref[...])
    h2 = jnp.maximum(h2, 0.0)
    acc_ref[...] += jnp.sum(h2, axis=0, keepdims=True)

    @pl.when(i == NBLK - 1)
    def _():
      hg = acc_ref[...] * (1.0 / N)
      o1 = jnp.maximum(
          jnp.dot(hg, wd_ref[...], preferred_element_type=jnp.float32)
          + bd_ref[...], 0.0)
      out_ref[...] = (
          jnp.dot(o1, wc_ref[...], preferred_element_type=jnp.float32)
          + bc_ref[...])

  return pl.pallas_call(
      body,
      grid=(NBLK,),
      in_specs=[
          pl.BlockSpec((BLK, HALF), lambda i: (i, 0)),
          pl.BlockSpec((BLK, HALF), lambda i: (i + NBLK, 0)),
          pl.BlockSpec((BLK, AUGW), lambda i: (i, 0)),
          pl.BlockSpec((HALF, D), lambda i: (0, 0)),
          pl.BlockSpec((HALF, D), lambda i: (1, 0)),
          pl.BlockSpec((1, D), lambda i: (0, 0)),
          pl.BlockSpec((D, HALF), lambda i: (0, 0)),
          pl.BlockSpec((1, HALF), lambda i: (0, 0)),
          pl.BlockSpec((HALF, 10), lambda i: (0, 0)),
          pl.BlockSpec((1, 10), lambda i: (0, 0)),
      ],
      out_specs=pl.BlockSpec((1, 10), lambda i: (0, 0)),
      out_shape=jax.ShapeDtypeStruct((1, 10), jnp.float32),
      scratch_shapes=[pltpu.VMEM((1, D), jnp.float32)],
      compiler_params=pltpu.CompilerParams(
          dimension_semantics=("arbitrary",)),
  )(a2, a2, a1, w2, w2, b2.reshape(1, D), wd, bd.reshape(1, HALF),
    wc, bc.reshape(1, 10))


def kernel(in_feat, edge_weights, W1, b1, W2, b2, Wd, bd, Wc, bc, edge_index):
  npad = E_PAD - E
  src = jnp.concatenate([edge_index[0], jnp.zeros((npad,), jnp.int32)])
  dst = jnp.concatenate([edge_index[1], jnp.zeros((npad,), jnp.int32)])
  w = jnp.concatenate([edge_weights, jnp.zeros((npad,), jnp.float32)])
  w_bits = lax.bitcast_convert_type(w, jnp.int32)
  packed = jnp.stack([src, src + N, dst, w_bits], axis=0)
  packed = packed.reshape(4, ROWS_PAD, B).transpose(1, 0, 2)

  ones = jnp.ones((N, 1), jnp.float32)
  pad = jnp.zeros((N, AUGW - HALF - 1), jnp.float32)
  table1 = jnp.concatenate([
      jnp.concatenate([in_feat[:, :HALF], ones, pad], axis=1),
      jnp.concatenate([in_feat[:, HALF:], ones, pad], axis=1),
  ], axis=0)

  zeros_aug = jnp.zeros((NPT, AUGW), jnp.float32)
  zeros_half = jnp.zeros((NPT, HALF), jnp.float32)

  a1 = _sc_agg_aug(table1, packed, zeros_aug)
  h = _tc_layer1(a1, W1, b1)
  a2 = _sc_agg_half(h, packed, zeros_half)
  return _tc_layer2(a2, a1, W2, b2, Wd, bd, Wc, bc)
